# Initial kernel scaffold; baseline (speedup 1.0000x reference)
import jax
import jax.numpy as jnp
from jax.experimental import pallas as pl


def kernel(features, edge_index_0, edge_index_1, edge_index_2, edge_weight_1, edge_weight_2, W0, b0, W1, b1, W2, b2):
    raise NotImplementedError("write your pallas kernel here")


# speedup vs baseline: 13.1810x; 13.1810x over previous
import dataclasses
import functools

import jax
import jax.numpy as jnp
from jax import lax
from jax.experimental import pallas as pl
from jax.experimental.pallas import tpu as pltpu
from jax.experimental.pallas import tpu_sc as plsc

_N = 10000
_E = 320000
_D_IN = 128
_D_HID = 128
_D_OUT = 64
_ALPHA = 0.5

_NC = 2
_NS = 16
_NW = _NC * _NS
_EPW = _E // _NW
_CH = 80
_NCHUNK = _EPW // _CH
_NPDEG = 10240
_NPROW = 10240
_BR = 1000


def _mesh():
    return plsc.VectorSubcoreMesh(core_axis_name="c", subcore_axis_name="s")


def _sc_params():
    cp = pltpu.CompilerParams()
    if "needs_layout_passes" in pltpu.CompilerParams.__dataclass_fields__:
        cp = dataclasses.replace(cp, needs_layout_passes=False)
    return cp


def _deg_body(col0, col1, col2, ew1, ew2, out, cbuf, ibuf, wbuf, zbuf, sacc):
    c = lax.axis_index("c")
    s = lax.axis_index("s")
    wid = c * _NS + s
    z16 = jnp.zeros((16,), jnp.float32)
    for i in range(8):
        zbuf[pl.ds(i * 16, 16)] = z16
    rbase = s * 1920

    @pl.loop(0, 15)
    def _zero(i):
        pltpu.sync_copy(zbuf, sacc.at[pl.ds(rbase + i * 128, 128)])

    plsc.subcore_barrier()

    one16 = jnp.ones((16,), jnp.float32)
    for i in range(_CH // 16):
        wbuf[pl.ds(i * 16, 16)] = one16

    for set_id, (colr, ewr) in enumerate(
        ((col0, None), (col1, ew1), (col2, ew2))
    ):
        off = set_id * _NPDEG

        @pl.loop(0, _NCHUNK)
        def _chunk(ci, colr=colr, ewr=ewr, off=off):
            base = wid * _EPW + ci * _CH
            pltpu.sync_copy(colr.at[pl.ds(base, _CH)], cbuf)
            if ewr is not None:
                pltpu.sync_copy(ewr.at[pl.ds(base, _CH)], wbuf)
            for i in range(_CH // 16):
                ibuf[pl.ds(i * 16, 16)] = cbuf[pl.ds(i * 16, 16)] + off
            pltpu.sync_copy(wbuf, sacc.at[ibuf], add=True)

    plsc.subcore_barrier()
    pltpu.sync_copy(sacc.at[pl.ds(rbase, 1920)], out.at[c, pl.ds(rbase, 1920)])


def _deg_call(col0, col1, col2, ew1, ew2):
    k = pl.kernel(
        _deg_body,
        out_type=jax.ShapeDtypeStruct((_NC, 3 * _NPDEG), jnp.float32),
        mesh=_mesh(),
        scratch_types=[
            pltpu.VMEM((_CH,), jnp.int32),
            pltpu.VMEM((_CH,), jnp.int32),
            pltpu.VMEM((_CH,), jnp.float32),
            pltpu.VMEM((128,), jnp.float32),
            pltpu.VMEM_SHARED((3 * _NPDEG,), jnp.float32),
        ],
    )
    return k(col0, col1, col2, ew1, ew2)


def _make_prop_body(d, weighted, zrows):
    def body(hn, rowr, colr, *rest):
        if weighted:
            ewr, out, rbuf, cbuf, wbuf, gbuf, zbuf, sacc = rest
        else:
            out, rbuf, cbuf, wbuf, gbuf, zbuf, sacc = rest
        c = lax.axis_index("c")
        s = lax.axis_index("s")
        wid = c * _NS + s
        z16 = jnp.zeros((16,), jnp.float32)
        for r in range(zrows):
            for f in range(d // 16):
                zbuf[r, pl.ds(f * 16, 16)] = z16
        rows_per_sub = _NPROW // _NS
        rbase = s * rows_per_sub

        @pl.loop(0, rows_per_sub // zrows)
        def _zero(i):
            pltpu.sync_copy(zbuf, sacc.at[pl.ds(rbase + i * zrows, zrows)])

        plsc.subcore_barrier()

        @pl.loop(0, _NCHUNK)
        def _chunk(ci):
            base = wid * _EPW + ci * _CH
            pltpu.sync_copy(rowr.at[pl.ds(base, _CH)], rbuf)
            pltpu.sync_copy(colr.at[pl.ds(base, _CH)], cbuf)
            if weighted:
                pltpu.sync_copy(ewr.at[pl.ds(base, _CH)], wbuf)
            pltpu.sync_copy(hn.at[rbuf], gbuf)

            if weighted:
                @pl.loop(0, _CH)
                def _scale(e):
                    ev = jnp.zeros((16,), jnp.int32) + e
                    we = plsc.load_gather(wbuf, [ev])
                    for f in range(d // 16):
                        sl = pl.ds(f * 16, 16)
                        gbuf[e, sl] = gbuf[e, sl] * we

            pltpu.sync_copy(gbuf, sacc.at[cbuf], add=True)

        plsc.subcore_barrier()
        pltpu.sync_copy(
            sacc.at[pl.ds(rbase, rows_per_sub)],
            out.at[c, pl.ds(rbase, rows_per_sub)],
        )

    return body


def _prop_call(hn, rowi, coli, ew, d):
    weighted = ew is not None
    zrows = 32
    k = pl.kernel(
        _make_prop_body(d, weighted, zrows),
        out_type=jax.ShapeDtypeStruct((_NC, _NPROW, d), jnp.float32),
        mesh=_mesh(),
        scratch_types=[
            pltpu.VMEM((_CH,), jnp.int32),
            pltpu.VMEM((_CH,), jnp.int32),
            pltpu.VMEM((_CH,), jnp.float32),
            pltpu.VMEM((_CH, d), jnp.float32),
            pltpu.VMEM((zrows, d), jnp.float32),
            pltpu.VMEM_SHARED((_NPROW, d), jnp.float32),
        ],
        compiler_params=_sc_params(),
    )
    if weighted:
        return k(hn, rowi, coli, ew)
    return k(hn, rowi, coli)


def _row_spec(d):
    return pl.BlockSpec((_BR, d), lambda j: (j, 0))


_DIS_SPEC = pl.BlockSpec((_BR, 3), lambda j: (j, 0))


def _full_spec(*dims):
    return pl.BlockSpec(dims, lambda j: tuple(0 for _ in dims))


def _prep_body(degp_ref, feat_ref, w0_ref, dis_ref, h0_ref, hn0_ref):
    deg = degp_ref[0] + degp_ref[1] + 1.0
    dis = jnp.where(deg > 0, lax.rsqrt(deg), 0.0)
    dis_ref[...] = dis
    h0 = jnp.dot(feat_ref[...], w0_ref[...], preferred_element_type=jnp.float32)
    h0_ref[...] = h0
    hn0_ref[...] = h0 * dis[:, 0][:, None]


def _prep_call(degp, features, w0):
    return pl.pallas_call(
        _prep_body,
        grid=(_N // _BR,),
        in_specs=[
            pl.BlockSpec((_NC, _BR, 3), lambda j: (0, j, 0)),
            _row_spec(_D_IN),
            _full_spec(_D_IN, _D_HID),
        ],
        out_specs=[_DIS_SPEC, _row_spec(_D_HID), _row_spec(_D_HID)],
        out_shape=[
            jax.ShapeDtypeStruct((_N, 3), jnp.float32),
            jax.ShapeDtypeStruct((_N, _D_HID), jnp.float32),
            jax.ShapeDtypeStruct((_N, _D_HID), jnp.float32),
        ],
    )(degp, features, w0)


def _mix0_body(accp_ref, h0_ref, dis_ref, b0_ref, w1_ref,
               x0_ref, h1_ref, hn1_ref):
    acc = accp_ref[0] + accp_ref[1]
    d0 = dis_ref[:, 0]
    d1 = dis_ref[:, 1]
    h0 = h0_ref[...]
    x0 = d0[:, None] * acc + (d0 * d0)[:, None] * h0 + b0_ref[...]
    x0_ref[...] = x0
    r = jnp.maximum(x0, 0.0)
    h1 = jnp.dot(r, w1_ref[...], preferred_element_type=jnp.float32)
    h1_ref[...] = h1
    hn1_ref[...] = h1 * d1[:, None]


def _mix0_call(accp, h0, dis, b0, w1):
    return pl.pallas_call(
        _mix0_body,
        grid=(_N // _BR,),
        in_specs=[
            pl.BlockSpec((_NC, _BR, _D_HID), lambda j: (0, j, 0)),
            _row_spec(_D_HID),
            _DIS_SPEC,
            _full_spec(1, _D_HID),
            _full_spec(_D_HID, _D_HID),
        ],
        out_specs=[_row_spec(_D_HID)] * 3,
        out_shape=[jax.ShapeDtypeStruct((_N, _D_HID), jnp.float32)] * 3,
    )(accp, h0, dis, b0, w1)


def _mix1_body(accp_ref, h1_ref, x0_ref, dis_ref, b1_ref, w2_ref,
               h2_ref, hn2_ref):
    acc = accp_ref[0] + accp_ref[1]
    d1 = dis_ref[:, 1]
    d2 = dis_ref[:, 2]
    h1 = h1_ref[...]
    g1 = d1[:, None] * acc + (d1 * d1)[:, None] * h1 + b1_ref[...]
    x1 = _ALPHA * g1 + (1.0 - _ALPHA) * x0_ref[...]
    r = jnp.maximum(x1, 0.0)
    h2 = jnp.dot(r, w2_ref[...], preferred_element_type=jnp.float32)
    h2_ref[...] = h2
    hn2_ref[...] = jnp.concatenate(
        [h2 * d2[:, None], jnp.zeros((_BR, _D_HID - _D_OUT), jnp.float32)],
        axis=1,
    )


def _mix1_call(accp, h1, x0, dis, b1, w2):
    return pl.pallas_call(
        _mix1_body,
        grid=(_N // _BR,),
        in_specs=[
            pl.BlockSpec((_NC, _BR, _D_HID), lambda j: (0, j, 0)),
            _row_spec(_D_HID),
            _row_spec(_D_HID),
            _DIS_SPEC,
            _full_spec(1, _D_HID),
            _full_spec(_D_HID, _D_OUT),
        ],
        out_specs=[_row_spec(_D_OUT), _row_spec(_D_HID)],
        out_shape=[
            jax.ShapeDtypeStruct((_N, _D_OUT), jnp.float32),
            jax.ShapeDtypeStruct((_N, _D_HID), jnp.float32),
        ],
    )(accp, h1, x0, dis, b1, w2)


def _final_body(accp_ref, h2_ref, dis_ref, b2_ref, out_ref):
    acc = (accp_ref[0] + accp_ref[1])[:, :_D_OUT]
    d2 = dis_ref[:, 2]
    g2 = d2[:, None] * acc + (d2 * d2)[:, None] * h2_ref[...] + b2_ref[...]
    m = jnp.max(g2, axis=1, keepdims=True)
    sh = g2 - m
    out_ref[...] = sh - jnp.log(jnp.sum(jnp.exp(sh), axis=1, keepdims=True))


def _final_call(accp, h2, dis, b2):
    return pl.pallas_call(
        _final_body,
        grid=(_N // _BR,),
        in_specs=[
            pl.BlockSpec((_NC, _BR, _D_HID), lambda j: (0, j, 0)),
            _row_spec(_D_OUT),
            _DIS_SPEC,
            _full_spec(1, _D_OUT),
        ],
        out_specs=_row_spec(_D_OUT),
        out_shape=jax.ShapeDtypeStruct((_N, _D_OUT), jnp.float32),
    )(accp, h2, dis, b2)


@jax.jit
def kernel(features, edge_index_0, edge_index_1, edge_index_2,
           edge_weight_1, edge_weight_2, W0, b0, W1, b1, W2, b2):
    row0 = edge_index_0[0].astype(jnp.int32)
    col0 = edge_index_0[1].astype(jnp.int32)
    row1 = edge_index_1[0].astype(jnp.int32)
    col1 = edge_index_1[1].astype(jnp.int32)
    row2 = edge_index_2[0].astype(jnp.int32)
    col2 = edge_index_2[1].astype(jnp.int32)
    ew1 = edge_weight_1.astype(jnp.float32)
    ew2 = edge_weight_2.astype(jnp.float32)

    degp = _deg_call(col0, col1, col2, ew1, ew2)
    degp = degp.reshape(_NC, 3, _NPDEG).transpose(0, 2, 1)

    h0, hn0, dis = None, None, None
    dis, h0, hn0 = _prep_call(degp, features, W0)

    acc0 = _prop_call(hn0, row0, col0, None, _D_HID)
    x0, h1, hn1 = _mix0_call(acc0, h0, dis, b0.reshape(1, _D_HID), W1)

    acc1 = _prop_call(hn1, row1, col1, ew1, _D_HID)
    h2, hn2 = _mix1_call(acc1, h1, x0, dis, b1.reshape(1, _D_HID), W2)

    acc2 = _prop_call(hn2, row2, col2, ew2, _D_HID)
    return _final_call(acc2, h2, dis, b2.reshape(1, _D_OUT))


# speedup vs baseline: 14.7683x; 1.1204x over previous
import dataclasses
import functools

import jax
import jax.numpy as jnp
from jax import lax
from jax.experimental import pallas as pl
from jax.experimental.pallas import tpu as pltpu
from jax.experimental.pallas import tpu_sc as plsc

_N = 10000
_E = 320000
_D_IN = 128
_D_HID = 128
_D_OUT = 64
_ALPHA = 0.5

_NC = 2
_NS = 16
_NW = _NC * _NS
_CH = 128
_CPW = 80
_EPW = _CPW * _CH
_E2 = _EPW * _NW
_TOTCH = _E2 // _CH
_NPDEG = 10240
_NPROW = 10240
_PADCOL = _N
_BR = 1000


def _mesh():
    return plsc.VectorSubcoreMesh(core_axis_name="c", subcore_axis_name="s")


def _sc_params():
    cp = pltpu.CompilerParams()
    if "needs_layout_passes" in pltpu.CompilerParams.__dataclass_fields__:
        cp = dataclasses.replace(cp, needs_layout_passes=False)
    return cp


def _deg_body(col0, col1, col2, ew1, ew2, out,
              i0, i1, i2, w1, w2, obuf, zbuf, sacc, sem):
    c = lax.axis_index("c")
    s = lax.axis_index("s")
    wid = c * _NS + s
    z16 = jnp.zeros((16,), jnp.float32)
    for i in range(8):
        zbuf[pl.ds(i * 16, 16)] = z16
    rbase = s * 1920

    @pl.loop(0, 15)
    def _zero(i):
        pltpu.sync_copy(zbuf, sacc.at[pl.ds(rbase + i * 128, 128)])

    one16 = jnp.ones((16,), jnp.float32)
    for i in range(8):
        obuf[pl.ds(i * 16, 16)] = one16

    rows = pl.ds(wid * _CPW, _CPW)
    pltpu.sync_copy(col0.at[rows], i0)
    pltpu.sync_copy(col1.at[rows], i1)
    pltpu.sync_copy(col2.at[rows], i2)
    pltpu.sync_copy(ew1.at[rows], w1)
    pltpu.sync_copy(ew2.at[rows], w2)

    for ib, off in ((i1, _NPDEG), (i2, 2 * _NPDEG)):
        @pl.loop(0, _CPW)
        def _shift(j, ib=ib, off=off):
            for f in range(8):
                sl = pl.ds(f * 16, 16)
                ib[j, sl] = ib[j, sl] + off

    plsc.subcore_barrier()

    for ib, wb in ((i0, None), (i1, w1), (i2, w2)):
        @pl.loop(0, _CPW)
        def _fire(j, ib=ib, wb=wb):
            src = obuf if wb is None else wb.at[j]
            pltpu.async_copy(src, sacc.at[ib.at[j]], sem, add=True)

    for ib, wb in ((i0, None), (i1, w1), (i2, w2)):
        @pl.loop(0, _CPW)
        def _drain(j, ib=ib, wb=wb):
            src = obuf if wb is None else wb.at[j]
            pltpu.make_async_copy(src, sacc.at[ib.at[j]], sem).wait()

    plsc.subcore_barrier()
    pltpu.sync_copy(sacc.at[pl.ds(rbase, 1920)], out.at[c, pl.ds(rbase, 1920)])


def _deg_call(col0, col1, col2, ew1, ew2):
    k = pl.kernel(
        _deg_body,
        out_type=jax.ShapeDtypeStruct((_NC, 3 * _NPDEG), jnp.float32),
        mesh=_mesh(),
        scratch_types=[
            pltpu.VMEM((_CPW, _CH), jnp.int32),
            pltpu.VMEM((_CPW, _CH), jnp.int32),
            pltpu.VMEM((_CPW, _CH), jnp.int32),
            pltpu.VMEM((_CPW, _CH), jnp.float32),
            pltpu.VMEM((_CPW, _CH), jnp.float32),
            pltpu.VMEM((_CH,), jnp.float32),
            pltpu.VMEM((_CH,), jnp.float32),
            pltpu.VMEM_SHARED((3 * _NPDEG,), jnp.float32),
            pltpu.SemaphoreType.DMA,
        ],
        compiler_params=_sc_params(),
    )
    return k(col0, col1, col2, ew1, ew2)


def _make_prop_body(d, weighted, zrows):
    def body(hn, rowp, colp, *rest):
        if weighted:
            ewp, out, ridx, cidx, widx, gbig, zbuf, sacc, si, sg, ss = rest
        else:
            out, ridx, cidx, widx, gbig, zbuf, sacc, si, sg, ss = rest
        c = lax.axis_index("c")
        s = lax.axis_index("s")
        wid = c * _NS + s
        z16 = jnp.zeros((16,), jnp.float32)
        for r in range(zrows):
            for f in range(d // 16):
                zbuf[r, pl.ds(f * 16, 16)] = z16
        rows_per_sub = _NPROW // _NS
        rbase = s * rows_per_sub

        @pl.loop(0, rows_per_sub // zrows)
        def _zero(i):
            pltpu.sync_copy(zbuf, sacc.at[pl.ds(rbase + i * zrows, zrows)])

        plsc.subcore_barrier()

        def gslice(j):
            return gbig.at[pl.ds((j % 2) * _CH, _CH)]

        def idx_copies(t):
            p = t % 3
            hrow = wid * _CPW + t
            res = [(rowp.at[hrow], ridx.at[p]), (colp.at[hrow], cidx.at[p])]
            if weighted:
                res.append((ewp.at[hrow], widx.at[p]))
            return res, si.at[p]

        def issue_idx(t):
            cps, sem = idx_copies(t)
            for src, dst in cps:
                pltpu.async_copy(src, dst, sem)

        def wait_idx(t):
            cps, sem = idx_copies(t)
            for src, dst in cps:
                pltpu.make_async_copy(src, dst, sem).wait()

        issue_idx(0)

        @pl.loop(0, _CPW + 2)
        def _ring(t):
            @pl.when(t >= 2)
            def _drain_s(t=t):
                j = t - 2
                pltpu.make_async_copy(
                    gslice(j), sacc.at[cidx.at[j % 3]], ss.at[j % 2]).wait()

            @pl.when(t < _CPW)
            def _issue_g(t=t):
                wait_idx(t)
                pltpu.async_copy(
                    hn.at[ridx.at[t % 3]], gslice(t), sg.at[t % 2])

            @pl.when(jnp.logical_and(t >= 1, t <= _CPW))
            def _work(t=t):
                j = t - 1
                p3 = j % 3
                p2 = j % 2
                pltpu.make_async_copy(
                    hn.at[ridx.at[p3]], gslice(j), sg.at[p2]).wait()
                if weighted:
                    pb = jnp.zeros((16,), jnp.int32) + p3
                    base = p2 * _CH

                    @pl.loop(0, _CH)
                    def _scale(e):
                        eb = jnp.zeros((16,), jnp.int32) + e
                        we = plsc.load_gather(widx, [pb, eb])
                        for f in range(d // 16):
                            sl = pl.ds(f * 16, 16)
                            gbig[base + e, sl] = gbig[base + e, sl] * we
                pltpu.async_copy(
                    gslice(j), sacc.at[cidx.at[p3]], ss.at[p2], add=True)

            @pl.when(t + 1 < _CPW)
            def _issue_i(t=t):
                issue_idx(t + 1)

        plsc.subcore_barrier()
        pltpu.sync_copy(
            sacc.at[pl.ds(rbase, rows_per_sub)],
            out.at[c, pl.ds(rbase, rows_per_sub)],
        )

    return body


def _prop_call(hn, rowp, colp, ewp, d):
    weighted = ewp is not None
    zrows = 32
    k = pl.kernel(
        _make_prop_body(d, weighted, zrows),
        out_type=jax.ShapeDtypeStruct((_NC, _NPROW, d), jnp.float32),
        mesh=_mesh(),
        scratch_types=[
            pltpu.VMEM((3, _CH), jnp.int32),
            pltpu.VMEM((3, _CH), jnp.int32),
            pltpu.VMEM((3, _CH), jnp.float32),
            pltpu.VMEM((2 * _CH, d), jnp.float32),
            pltpu.VMEM((zrows, d), jnp.float32),
            pltpu.VMEM_SHARED((_NPROW, d), jnp.float32),
            pltpu.SemaphoreType.DMA((3,)),
            pltpu.SemaphoreType.DMA((2,)),
            pltpu.SemaphoreType.DMA((2,)),
        ],
        compiler_params=_sc_params(),
    )
    if weighted:
        return k(hn, rowp, colp, ewp)
    return k(hn, rowp, colp)


def _row_spec(d):
    return pl.BlockSpec((_BR, d), lambda j: (j, 0))


_DIS_SPEC = pl.BlockSpec((_BR, 3), lambda j: (j, 0))


def _full_spec(*dims):
    return pl.BlockSpec(dims, lambda j: tuple(0 for _ in dims))


def _h0_body(feat_ref, w0_ref, h0_ref):
    h0_ref[...] = jnp.dot(feat_ref[...], w0_ref[...],
                          preferred_element_type=jnp.float32)


def _h0_call(features, w0):
    return pl.pallas_call(
        _h0_body,
        grid=(_N // _BR,),
        in_specs=[_row_spec(_D_IN), _full_spec(_D_IN, _D_HID)],
        out_specs=_row_spec(_D_HID),
        out_shape=jax.ShapeDtypeStruct((_N, _D_HID), jnp.float32),
    )(features, w0)


def _prep_body(degp_ref, h0_ref, dis_ref, hn0_ref):
    deg = degp_ref[0] + degp_ref[1] + 1.0
    dis = jnp.where(deg > 0, lax.rsqrt(deg), 0.0)
    dis_ref[...] = dis
    hn0_ref[...] = h0_ref[...] * dis[:, 0][:, None]


def _prep_call(degp, h0):
    return pl.pallas_call(
        _prep_body,
        grid=(_N // _BR,),
        in_specs=[
            pl.BlockSpec((_NC, _BR, 3), lambda j: (0, j, 0)),
            _row_spec(_D_HID),
        ],
        out_specs=[_DIS_SPEC, _row_spec(_D_HID)],
        out_shape=[
            jax.ShapeDtypeStruct((_N, 3), jnp.float32),
            jax.ShapeDtypeStruct((_N, _D_HID), jnp.float32),
        ],
    )(degp, h0)


def _mix0_body(accp_ref, h0_ref, dis_ref, b0_ref, w1_ref,
               x0_ref, h1_ref, hn1_ref):
    acc = accp_ref[0] + accp_ref[1]
    d0 = dis_ref[:, 0]
    d1 = dis_ref[:, 1]
    h0 = h0_ref[...]
    x0 = d0[:, None] * acc + (d0 * d0)[:, None] * h0 + b0_ref[...]
    x0_ref[...] = x0
    r = jnp.maximum(x0, 0.0)
    h1 = jnp.dot(r, w1_ref[...], preferred_element_type=jnp.float32)
    h1_ref[...] = h1
    hn1_ref[...] = h1 * d1[:, None]


def _mix0_call(accp, h0, dis, b0, w1):
    return pl.pallas_call(
        _mix0_body,
        grid=(_N // _BR,),
        in_specs=[
            pl.BlockSpec((_NC, _BR, _D_HID), lambda j: (0, j, 0)),
            _row_spec(_D_HID),
            _DIS_SPEC,
            _full_spec(1, _D_HID),
            _full_spec(_D_HID, _D_HID),
        ],
        out_specs=[_row_spec(_D_HID)] * 3,
        out_shape=[jax.ShapeDtypeStruct((_N, _D_HID), jnp.float32)] * 3,
    )(accp, h0, dis, b0, w1)


def _mix1_body(accp_ref, h1_ref, x0_ref, dis_ref, b1_ref, w2_ref,
               h2_ref, hn2_ref):
    acc = accp_ref[0] + accp_ref[1]
    d1 = dis_ref[:, 1]
    d2 = dis_ref[:, 2]
    h1 = h1_ref[...]
    g1 = d1[:, None] * acc + (d1 * d1)[:, None] * h1 + b1_ref[...]
    x1 = _ALPHA * g1 + (1.0 - _ALPHA) * x0_ref[...]
    r = jnp.maximum(x1, 0.0)
    h2 = jnp.dot(r, w2_ref[...], preferred_element_type=jnp.float32)
    h2_ref[...] = h2
    hn2_ref[...] = jnp.concatenate(
        [h2 * d2[:, None], jnp.zeros((_BR, _D_HID - _D_OUT), jnp.float32)],
        axis=1,
    )


def _mix1_call(accp, h1, x0, dis, b1, w2):
    return pl.pallas_call(
        _mix1_body,
        grid=(_N // _BR,),
        in_specs=[
            pl.BlockSpec((_NC, _BR, _D_HID), lambda j: (0, j, 0)),
            _row_spec(_D_HID),
            _row_spec(_D_HID),
            _DIS_SPEC,
            _full_spec(1, _D_HID),
            _full_spec(_D_HID, _D_OUT),
        ],
        out_specs=[_row_spec(_D_OUT), _row_spec(_D_HID)],
        out_shape=[
            jax.ShapeDtypeStruct((_N, _D_OUT), jnp.float32),
            jax.ShapeDtypeStruct((_N, _D_HID), jnp.float32),
        ],
    )(accp, h1, x0, dis, b1, w2)


def _final_body(accp_ref, h2_ref, dis_ref, b2_ref, out_ref):
    acc = (accp_ref[0] + accp_ref[1])[:, :_D_OUT]
    d2 = dis_ref[:, 2]
    g2 = d2[:, None] * acc + (d2 * d2)[:, None] * h2_ref[...] + b2_ref[...]
    m = jnp.max(g2, axis=1, keepdims=True)
    sh = g2 - m
    out_ref[...] = sh - jnp.log(jnp.sum(jnp.exp(sh), axis=1, keepdims=True))


def _final_call(accp, h2, dis, b2):
    return pl.pallas_call(
        _final_body,
        grid=(_N // _BR,),
        in_specs=[
            pl.BlockSpec((_NC, _BR, _D_HID), lambda j: (0, j, 0)),
            _row_spec(_D_OUT),
            _DIS_SPEC,
            _full_spec(1, _D_OUT),
        ],
        out_specs=_row_spec(_D_OUT),
        out_shape=jax.ShapeDtypeStruct((_N, _D_OUT), jnp.float32),
    )(accp, h2, dis, b2)


def _pad_chunks(a, pad_val, dtype):
    a = a.astype(dtype)
    pad = jnp.full((_E2 - _E,), pad_val, dtype)
    return jnp.concatenate([a, pad]).reshape(_TOTCH, _CH)


@jax.jit
def kernel(features, edge_index_0, edge_index_1, edge_index_2,
           edge_weight_1, edge_weight_2, W0, b0, W1, b1, W2, b2):
    row0 = _pad_chunks(edge_index_0[0], 0, jnp.int32)
    col0 = _pad_chunks(edge_index_0[1], _PADCOL, jnp.int32)
    row1 = _pad_chunks(edge_index_1[0], 0, jnp.int32)
    col1 = _pad_chunks(edge_index_1[1], _PADCOL, jnp.int32)
    row2 = _pad_chunks(edge_index_2[0], 0, jnp.int32)
    col2 = _pad_chunks(edge_index_2[1], _PADCOL, jnp.int32)
    ew1 = _pad_chunks(edge_weight_1, 0.0, jnp.float32)
    ew2 = _pad_chunks(edge_weight_2, 0.0, jnp.float32)

    degp = _deg_call(col0, col1, col2, ew1, ew2)
    degp = degp.reshape(_NC, 3, _NPDEG).transpose(0, 2, 1)

    h0 = _h0_call(features, W0)
    dis, hn0 = _prep_call(degp, h0)

    acc0 = _prop_call(hn0, row0, col0, None, _D_HID)
    x0, h1, hn1 = _mix0_call(acc0, h0, dis, b0.reshape(1, _D_HID), W1)

    acc1 = _prop_call(hn1, row1, col1, ew1, _D_HID)
    h2, hn2 = _mix1_call(acc1, h1, x0, dis, b1.reshape(1, _D_HID), W2)

    acc2 = _prop_call(hn2, row2, col2, ew2, _D_HID)
    return _final_call(acc2, h2, dis, b2.reshape(1, _D_OUT))


# speedup vs baseline: 15.7588x; 1.0671x over previous
import dataclasses
import functools

import jax
import jax.numpy as jnp
from jax import lax
from jax.experimental import pallas as pl
from jax.experimental.pallas import tpu as pltpu
from jax.experimental.pallas import tpu_sc as plsc

_N = 10000
_E = 320000
_D_IN = 128
_D_HID = 128
_D_OUT = 64
_ALPHA = 0.5

_NC = 2
_NS = 16
_NW = _NC * _NS
_CH = 128
_CPW = 80
_CPW0 = 120
_CPW1 = 40
_EPW = _CPW * _CH
_E2 = _EPW * _NW
_TOTCH = _E2 // _CH
_NPDEG = 10240
_NPROW = 10240
_PADCOL = _N
_BR = 1000


def _mesh():
    return plsc.VectorSubcoreMesh(core_axis_name="c", subcore_axis_name="s")


def _sc_params():
    cp = pltpu.CompilerParams()
    if "needs_layout_passes" in pltpu.CompilerParams.__dataclass_fields__:
        cp = dataclasses.replace(cp, needs_layout_passes=False)
    return cp


def _deg_body(col0, col1, col2, ew1, ew2, out,
              i0, i1, i2, w1, w2, obuf, zbuf, sacc, sem):
    c = lax.axis_index("c")
    s = lax.axis_index("s")
    wid = c * _NS + s
    z16 = jnp.zeros((16,), jnp.float32)
    for i in range(8):
        zbuf[pl.ds(i * 16, 16)] = z16
    rbase = s * 1920

    @pl.loop(0, 15)
    def _zero(i):
        pltpu.sync_copy(zbuf, sacc.at[pl.ds(rbase + i * 128, 128)])

    one16 = jnp.ones((16,), jnp.float32)
    for i in range(8):
        obuf[pl.ds(i * 16, 16)] = one16

    rows = pl.ds(wid * _CPW, _CPW)
    pltpu.sync_copy(col0.at[rows], i0)
    pltpu.sync_copy(col1.at[rows], i1)
    pltpu.sync_copy(col2.at[rows], i2)
    pltpu.sync_copy(ew1.at[rows], w1)
    pltpu.sync_copy(ew2.at[rows], w2)

    for ib, off in ((i1, _NPDEG), (i2, 2 * _NPDEG)):
        @pl.loop(0, _CPW)
        def _shift(j, ib=ib, off=off):
            for f in range(8):
                sl = pl.ds(f * 16, 16)
                ib[j, sl] = ib[j, sl] + off

    plsc.subcore_barrier()

    for ib, wb in ((i0, None), (i1, w1), (i2, w2)):
        @pl.loop(0, _CPW)
        def _fire(j, ib=ib, wb=wb):
            src = obuf if wb is None else wb.at[j]
            pltpu.async_copy(src, sacc.at[ib.at[j]], sem, add=True)

    for ib, wb in ((i0, None), (i1, w1), (i2, w2)):
        @pl.loop(0, _CPW)
        def _drain(j, ib=ib, wb=wb):
            src = obuf if wb is None else wb.at[j]
            pltpu.make_async_copy(src, sacc.at[ib.at[j]], sem).wait()

    plsc.subcore_barrier()
    pltpu.sync_copy(sacc.at[pl.ds(rbase, 1920)], out.at[c, pl.ds(rbase, 1920)])


def _deg_call(col0, col1, col2, ew1, ew2):
    k = pl.kernel(
        _deg_body,
        out_type=jax.ShapeDtypeStruct((_NC, 3 * _NPDEG), jnp.float32),
        mesh=_mesh(),
        scratch_types=[
            pltpu.VMEM((_CPW, _CH), jnp.int32),
            pltpu.VMEM((_CPW, _CH), jnp.int32),
            pltpu.VMEM((_CPW, _CH), jnp.int32),
            pltpu.VMEM((_CPW, _CH), jnp.float32),
            pltpu.VMEM((_CPW, _CH), jnp.float32),
            pltpu.VMEM((_CH,), jnp.float32),
            pltpu.VMEM((_CH,), jnp.float32),
            pltpu.VMEM_SHARED((3 * _NPDEG,), jnp.float32),
            pltpu.SemaphoreType.DMA,
        ],
        compiler_params=_sc_params(),
    )
    return k(col0, col1, col2, ew1, ew2)


def _make_prop_body(d, weighted, zrows):
    def body(hn, rowp, colp, *rest):
        if weighted:
            ewp, out, ridx, cidx, widx, gbig, zbuf, sacc, si, sg, ss = rest
        else:
            out, ridx, cidx, widx, gbig, zbuf, sacc, si, sg, ss = rest
        c = lax.axis_index("c")
        s = lax.axis_index("s")
        wid = c * _NS + s
        z16 = jnp.zeros((16,), jnp.float32)
        for r in range(zrows):
            for f in range(d // 16):
                zbuf[r, pl.ds(f * 16, 16)] = z16
        rows_per_sub = _NPROW // _NS
        rbase = s * rows_per_sub

        @pl.loop(0, rows_per_sub // zrows)
        def _zero(i):
            pltpu.sync_copy(zbuf, sacc.at[pl.ds(rbase + i * zrows, zrows)])

        plsc.subcore_barrier()

        def gslice(j):
            return gbig.at[pl.ds((j % 2) * _CH, _CH)]

        def run_ring(cnt, hbase):
            def idx_copies(t):
                p = t % 3
                hrow = hbase + t
                res = [(rowp.at[hrow], ridx.at[p]),
                       (colp.at[hrow], cidx.at[p])]
                if weighted:
                    res.append((ewp.at[hrow], widx.at[p]))
                return res, si.at[p]

            def issue_idx(t):
                cps, sem = idx_copies(t)
                for src, dst in cps:
                    pltpu.async_copy(src, dst, sem)

            def wait_idx(t):
                cps, sem = idx_copies(t)
                for src, dst in cps:
                    pltpu.make_async_copy(src, dst, sem).wait()

            issue_idx(0)

            @pl.loop(0, cnt + 2)
            def _ring(t):
                @pl.when(t >= 2)
                def _drain_s(t=t):
                    j = t - 2
                    pltpu.make_async_copy(
                        gslice(j), sacc.at[cidx.at[j % 3]],
                        ss.at[j % 2]).wait()

                @pl.when(t < cnt)
                def _issue_g(t=t):
                    wait_idx(t)
                    pltpu.async_copy(
                        hn.at[ridx.at[t % 3]], gslice(t), sg.at[t % 2])

                @pl.when(jnp.logical_and(t >= 1, t <= cnt))
                def _work(t=t):
                    j = t - 1
                    p3 = j % 3
                    p2 = j % 2
                    pltpu.make_async_copy(
                        hn.at[ridx.at[p3]], gslice(j), sg.at[p2]).wait()
                    if weighted:
                        pb = jnp.zeros((16,), jnp.int32) + p3
                        base = p2 * _CH

                        @pl.loop(0, _CH)
                        def _scale(e):
                            eb = jnp.zeros((16,), jnp.int32) + e
                            we = plsc.load_gather(widx, [pb, eb])
                            for f in range(d // 16):
                                sl = pl.ds(f * 16, 16)
                                gbig[base + e, sl] = gbig[base + e, sl] * we
                    pltpu.async_copy(
                        gslice(j), sacc.at[cidx.at[p3]], ss.at[p2], add=True)

                @pl.when(t + 1 < cnt)
                def _issue_i(t=t):
                    issue_idx(t + 1)

        @pl.when(c == 0)
        def _core0():
            run_ring(_CPW0, s * _CPW0)

        @pl.when(c == 1)
        def _core1():
            run_ring(_CPW1, _NS * _CPW0 + s * _CPW1)

        plsc.subcore_barrier()
        pltpu.sync_copy(
            sacc.at[pl.ds(rbase, rows_per_sub)],
            out.at[c, pl.ds(rbase, rows_per_sub)],
        )

    return body


def _prop_call(hn, rowp, colp, ewp, d):
    weighted = ewp is not None
    zrows = 32
    k = pl.kernel(
        _make_prop_body(d, weighted, zrows),
        out_type=jax.ShapeDtypeStruct((_NC, _NPROW, d), jnp.float32),
        mesh=_mesh(),
        scratch_types=[
            pltpu.VMEM((3, _CH), jnp.int32),
            pltpu.VMEM((3, _CH), jnp.int32),
            pltpu.VMEM((3, _CH), jnp.float32),
            pltpu.VMEM((2 * _CH, d), jnp.float32),
            pltpu.VMEM((zrows, d), jnp.float32),
            pltpu.VMEM_SHARED((_NPROW, d), jnp.float32),
            pltpu.SemaphoreType.DMA((3,)),
            pltpu.SemaphoreType.DMA((2,)),
            pltpu.SemaphoreType.DMA((2,)),
        ],
        compiler_params=_sc_params(),
    )
    if weighted:
        return k(hn, rowp, colp, ewp)
    return k(hn, rowp, colp)


def _row_spec(d):
    return pl.BlockSpec((_BR, d), lambda j: (j, 0))


_DIS_SPEC = pl.BlockSpec((_BR, 3), lambda j: (j, 0))


def _full_spec(*dims):
    return pl.BlockSpec(dims, lambda j: tuple(0 for _ in dims))


def _h0_body(feat_ref, w0_ref, h0_ref):
    h0_ref[...] = jnp.dot(feat_ref[...], w0_ref[...],
                          preferred_element_type=jnp.float32)


def _h0_call(features, w0):
    return pl.pallas_call(
        _h0_body,
        grid=(_N // _BR,),
        in_specs=[_row_spec(_D_IN), _full_spec(_D_IN, _D_HID)],
        out_specs=_row_spec(_D_HID),
        out_shape=jax.ShapeDtypeStruct((_N, _D_HID), jnp.float32),
    )(features, w0)


def _prep_body(degp_ref, h0_ref, dis_ref, hn0_ref):
    deg = degp_ref[0] + degp_ref[1] + 1.0
    dis = jnp.where(deg > 0, lax.rsqrt(deg), 0.0)
    dis_ref[...] = dis
    hn0_ref[...] = h0_ref[...] * dis[:, 0][:, None]


def _prep_call(degp, h0):
    return pl.pallas_call(
        _prep_body,
        grid=(_N // _BR,),
        in_specs=[
            pl.BlockSpec((_NC, _BR, 3), lambda j: (0, j, 0)),
            _row_spec(_D_HID),
        ],
        out_specs=[_DIS_SPEC, _row_spec(_D_HID)],
        out_shape=[
            jax.ShapeDtypeStruct((_N, 3), jnp.float32),
            jax.ShapeDtypeStruct((_N, _D_HID), jnp.float32),
        ],
    )(degp, h0)


def _mix0_body(accp_ref, h0_ref, dis_ref, b0_ref, w1_ref,
               x0_ref, h1_ref, hn1_ref):
    acc = accp_ref[0] + accp_ref[1]
    d0 = dis_ref[:, 0]
    d1 = dis_ref[:, 1]
    h0 = h0_ref[...]
    x0 = d0[:, None] * acc + (d0 * d0)[:, None] * h0 + b0_ref[...]
    x0_ref[...] = x0
    r = jnp.maximum(x0, 0.0)
    h1 = jnp.dot(r, w1_ref[...], preferred_element_type=jnp.float32)
    h1_ref[...] = h1
    hn1_ref[...] = h1 * d1[:, None]


def _mix0_call(accp, h0, dis, b0, w1):
    return pl.pallas_call(
        _mix0_body,
        grid=(_N // _BR,),
        in_specs=[
            pl.BlockSpec((_NC, _BR, _D_HID), lambda j: (0, j, 0)),
            _row_spec(_D_HID),
            _DIS_SPEC,
            _full_spec(1, _D_HID),
            _full_spec(_D_HID, _D_HID),
        ],
        out_specs=[_row_spec(_D_HID)] * 3,
        out_shape=[jax.ShapeDtypeStruct((_N, _D_HID), jnp.float32)] * 3,
    )(accp, h0, dis, b0, w1)


def _mix1_body(accp_ref, h1_ref, x0_ref, dis_ref, b1_ref, w2_ref,
               h2_ref, hn2_ref):
    acc = accp_ref[0] + accp_ref[1]
    d1 = dis_ref[:, 1]
    d2 = dis_ref[:, 2]
    h1 = h1_ref[...]
    g1 = d1[:, None] * acc + (d1 * d1)[:, None] * h1 + b1_ref[...]
    x1 = _ALPHA * g1 + (1.0 - _ALPHA) * x0_ref[...]
    r = jnp.maximum(x1, 0.0)
    h2 = jnp.dot(r, w2_ref[...], preferred_element_type=jnp.float32)
    h2_ref[...] = h2
    hn2_ref[...] = jnp.concatenate(
        [h2 * d2[:, None], jnp.zeros((_BR, _D_HID - _D_OUT), jnp.float32)],
        axis=1,
    )


def _mix1_call(accp, h1, x0, dis, b1, w2):
    return pl.pallas_call(
        _mix1_body,
        grid=(_N // _BR,),
        in_specs=[
            pl.BlockSpec((_NC, _BR, _D_HID), lambda j: (0, j, 0)),
            _row_spec(_D_HID),
            _row_spec(_D_HID),
            _DIS_SPEC,
            _full_spec(1, _D_HID),
            _full_spec(_D_HID, _D_OUT),
        ],
        out_specs=[_row_spec(_D_OUT), _row_spec(_D_HID)],
        out_shape=[
            jax.ShapeDtypeStruct((_N, _D_OUT), jnp.float32),
            jax.ShapeDtypeStruct((_N, _D_HID), jnp.float32),
        ],
    )(accp, h1, x0, dis, b1, w2)


def _final_body(accp_ref, h2_ref, dis_ref, b2_ref, out_ref):
    acc = (accp_ref[0] + accp_ref[1])[:, :_D_OUT]
    d2 = dis_ref[:, 2]
    g2 = d2[:, None] * acc + (d2 * d2)[:, None] * h2_ref[...] + b2_ref[...]
    m = jnp.max(g2, axis=1, keepdims=True)
    sh = g2 - m
    out_ref[...] = sh - jnp.log(jnp.sum(jnp.exp(sh), axis=1, keepdims=True))


def _final_call(accp, h2, dis, b2):
    return pl.pallas_call(
        _final_body,
        grid=(_N // _BR,),
        in_specs=[
            pl.BlockSpec((_NC, _BR, _D_HID), lambda j: (0, j, 0)),
            _row_spec(_D_OUT),
            _DIS_SPEC,
            _full_spec(1, _D_OUT),
        ],
        out_specs=_row_spec(_D_OUT),
        out_shape=jax.ShapeDtypeStruct((_N, _D_OUT), jnp.float32),
    )(accp, h2, dis, b2)


def _pad_chunks(a, pad_val, dtype):
    a = a.astype(dtype)
    pad = jnp.full((_E2 - _E,), pad_val, dtype)
    return jnp.concatenate([a, pad]).reshape(_TOTCH, _CH)


@jax.jit
def kernel(features, edge_index_0, edge_index_1, edge_index_2,
           edge_weight_1, edge_weight_2, W0, b0, W1, b1, W2, b2):
    row0 = _pad_chunks(edge_index_0[0], 0, jnp.int32)
    col0 = _pad_chunks(edge_index_0[1], _PADCOL, jnp.int32)
    row1 = _pad_chunks(edge_index_1[0], 0, jnp.int32)
    col1 = _pad_chunks(edge_index_1[1], _PADCOL, jnp.int32)
    row2 = _pad_chunks(edge_index_2[0], 0, jnp.int32)
    col2 = _pad_chunks(edge_index_2[1], _PADCOL, jnp.int32)
    ew1 = _pad_chunks(edge_weight_1, 0.0, jnp.float32)
    ew2 = _pad_chunks(edge_weight_2, 0.0, jnp.float32)

    degp = _deg_call(col0, col1, col2, ew1, ew2)
    degp = degp.reshape(_NC, 3, _NPDEG).transpose(0, 2, 1)

    h0 = _h0_call(features, W0)
    dis, hn0 = _prep_call(degp, h0)

    acc0 = _prop_call(hn0, row0, col0, None, _D_HID)
    x0, h1, hn1 = _mix0_call(acc0, h0, dis, b0.reshape(1, _D_HID), W1)

    acc1 = _prop_call(hn1, row1, col1, ew1, _D_HID)
    h2, hn2 = _mix1_call(acc1, h1, x0, dis, b1.reshape(1, _D_HID), W2)

    acc2 = _prop_call(hn2, row2, col2, ew2, _D_HID)
    return _final_call(acc2, h2, dis, b2.reshape(1, _D_OUT))
